# baseline (speedup 1.0000x reference)
import jax
import jax.numpy as jnp
from jax import lax
from jax.experimental import pallas as pl
from jax.experimental.pallas import tpu as pltpu
from jax.experimental.pallas import tpu_sc as plsc

BATCH = 16384
HIST = 50
EMBED_DIM = 32
NUM_FLAT = BATCH * HIST

_info = plsc.get_sparse_core_info()
_NC, _NS = _info.num_cores, _info.num_subcores
_NW = _NC * _NS
_B_PER_W = NUM_FLAT // _NW
_CHUNK = 800
_NSLOT = 4
_LOOK = 2
_NCHUNK = _B_PER_W // _CHUNK
_NGRP = _NCHUNK // _NSLOT


def _gather_body(idx_hbm, table_hbm, out_hbm, idx_v, rows_v, gsem, wsem):
    wid = lax.axis_index("s") * _NC + lax.axis_index("c")
    base = wid * _B_PER_W

    def fire(c, s):
        pltpu.sync_copy(idx_hbm.at[pl.ds(base + c * _CHUNK, _CHUNK)],
                        idx_v.at[s])
        pltpu.async_copy(table_hbm.at[idx_v.at[s]], rows_v.at[s], gsem.at[s])

    def wait_gather(s):
        pltpu.make_async_copy(table_hbm.at[idx_v.at[s]], rows_v.at[s],
                              gsem.at[s]).wait()

    def start_wb(c, s):
        pltpu.async_copy(rows_v.at[s],
                         out_hbm.at[pl.ds(base + c * _CHUNK, _CHUNK)],
                         wsem.at[s])

    def wait_wb(c, s):
        pltpu.make_async_copy(rows_v.at[s],
                              out_hbm.at[pl.ds(base + c * _CHUNK, _CHUNK)],
                              wsem.at[s]).wait()

    for c in range(_LOOK):
        fire(c, c)

    def group(j, carry):
        c0 = j * _NSLOT
        for b in range(_NSLOT):
            c = c0 + b
            wait_gather(b)
            start_wb(c, b)
            d = c + _LOOK
            sd = (b + _LOOK) % _NSLOT

            @pl.when(c >= _LOOK)
            def _free():
                wait_wb(c - _LOOK, sd)

            @pl.when(d < _NCHUNK)
            def _refill():
                fire(d, sd)

        return carry

    lax.fori_loop(0, _NGRP, group, 0)

    for k in range(_LOOK):
        c = _NCHUNK - _LOOK + k
        wait_wb(c, c % _NSLOT)


@jax.jit
def _gather(idx_flat, table):
    mesh = plsc.VectorSubcoreMesh(core_axis_name="c", subcore_axis_name="s")
    return pl.kernel(
        _gather_body,
        out_type=jax.ShapeDtypeStruct((NUM_FLAT, EMBED_DIM), jnp.float32),
        mesh=mesh,
        scratch_types=[
            pltpu.VMEM((_NSLOT, _CHUNK), jnp.int32),
            pltpu.VMEM((_NSLOT, _CHUNK, EMBED_DIM), jnp.float32),
            pltpu.SemaphoreType.DMA((_NSLOT,)),
            pltpu.SemaphoreType.DMA((_NSLOT,)),
        ],
        compiler_params=pltpu.CompilerParams(use_tc_tiling_on_sc=False),
    )(idx_flat, table)


def _tc_dummy_body(t_ref, o_ref):
    o_ref[...] = t_ref[...] * 1.000001


@jax.jit
def _tc_dummy(table):
    return pl.pallas_call(
        _tc_dummy_body,
        out_shape=jax.ShapeDtypeStruct((8000, EMBED_DIM), jnp.float32),
        grid=(625,),
        in_specs=[pl.BlockSpec((8000, EMBED_DIM), lambda i: (i % 125, 0))],
        out_specs=pl.BlockSpec((8000, EMBED_DIM), lambda i: (0, 0)),
    )(table)


def kernel(idx, table):
    idx_flat = idx.reshape(NUM_FLAT).astype(jnp.int32)
    out = _gather(idx_flat, table)
    probe = _tc_dummy(table)
    out = out + 0.0 * probe[0, 0]
    return out.reshape(BATCH, HIST, EMBED_DIM)


# speedup vs baseline: 1.7435x; 1.7435x over previous
import jax
import jax.numpy as jnp
from jax import lax
from jax.experimental import pallas as pl
from jax.experimental.pallas import tpu as pltpu
from jax.experimental.pallas import tpu_sc as plsc

BATCH = 16384
HIST = 50
EMBED_DIM = 32
NUM_FLAT = BATCH * HIST

_info = plsc.get_sparse_core_info()
_NC, _NS = _info.num_cores, _info.num_subcores
_NW = _NC * _NS
_B_PER_W = NUM_FLAT // _NW
_CHUNK = 800
_NSLOT = 4
_LOOK = 2
_NCHUNK = _B_PER_W // _CHUNK
_NGRP = _NCHUNK // _NSLOT


def _gather_body(idx_hbm, table_hbm, out_hbm, idx_v, rows_v, gsem, wsem):
    wid = lax.axis_index("s") * _NC + lax.axis_index("c")
    base = wid * _B_PER_W

    def fire(c, s):
        pltpu.sync_copy(idx_hbm.at[pl.ds(base + c * _CHUNK, _CHUNK)],
                        idx_v.at[s])
        pltpu.async_copy(table_hbm.at[idx_v.at[s]], rows_v.at[s], gsem.at[s])

    def wait_gather(s):
        pltpu.make_async_copy(table_hbm.at[idx_v.at[s]], rows_v.at[s],
                              gsem.at[s]).wait()

    def start_wb(c, s):
        pltpu.async_copy(rows_v.at[s],
                         out_hbm.at[pl.ds(base + c * _CHUNK, _CHUNK)],
                         wsem.at[s])

    def wait_wb(c, s):
        pltpu.make_async_copy(rows_v.at[s],
                              out_hbm.at[pl.ds(base + c * _CHUNK, _CHUNK)],
                              wsem.at[s]).wait()

    for c in range(_LOOK):
        fire(c, c)

    def group(j, carry):
        c0 = j * _NSLOT
        for b in range(_NSLOT):
            c = c0 + b
            wait_gather(b)
            start_wb(c, b)
            d = c + _LOOK
            sd = (b + _LOOK) % _NSLOT

            @pl.when(c >= _LOOK)
            def _free():
                wait_wb(c - _LOOK, sd)

            @pl.when(d < _NCHUNK)
            def _refill():
                fire(d, sd)

        return carry

    lax.fori_loop(0, _NGRP, group, 0)

    for k in range(_LOOK):
        c = _NCHUNK - _LOOK + k
        wait_wb(c, c % _NSLOT)


@jax.jit
def _gather(idx_flat, table):
    mesh = plsc.VectorSubcoreMesh(core_axis_name="c", subcore_axis_name="s")
    return pl.kernel(
        _gather_body,
        out_type=jax.ShapeDtypeStruct((NUM_FLAT, EMBED_DIM), jnp.float32),
        mesh=mesh,
        scratch_types=[
            pltpu.VMEM((_NSLOT, _CHUNK), jnp.int32),
            pltpu.VMEM((_NSLOT, _CHUNK, EMBED_DIM), jnp.float32),
            pltpu.SemaphoreType.DMA((_NSLOT,)),
            pltpu.SemaphoreType.DMA((_NSLOT,)),
        ],
        compiler_params=pltpu.CompilerParams(use_tc_tiling_on_sc=False),
    )(idx_flat, table)


def kernel(idx, table):
    idx_flat = idx.reshape(NUM_FLAT).astype(jnp.int32)
    out = _gather(idx_flat, table)
    return out.reshape(BATCH, HIST, EMBED_DIM)
